# baseline (speedup 1.0000x reference)
import jax
import jax.numpy as jnp
from jax import lax
from jax.experimental import pallas as pl
from jax.experimental.pallas import tpu as pltpu
from jax.experimental.pallas import tpu_sc as plsc

VOCAB = 1000000
DIM = 64
B = 16384
SUB = 8

NC = 2
NS = 16
L = 16
NW = NC * NS
BPW = B // NW
CH = 32
NCH = BPW // CH

_LN2 = 0.6931471805599453
_SQRT2 = 1.4142135623730951


def _ln(x):
    bits = plsc.bitcast(x, jnp.int32)
    e = ((bits >> 23) & 0xFF) - 127
    m = plsc.bitcast((bits & 0x007FFFFF) | 0x3F800000, jnp.float32)
    big = m > _SQRT2
    m = jnp.where(big, m * 0.5, m)
    e = (e + jnp.where(big, 1, 0)).astype(jnp.float32)
    z = (m - 1.0) / (m + 1.0)
    z2 = z * z
    poly = 1.0 + z2 * (1.0 / 3.0 + z2 * (1.0 / 5.0 + z2 * (1.0 / 7.0 + z2 * (1.0 / 9.0))))
    return 2.0 * z * poly + e * _LN2


NSEM = 4


def _sc_body(cq_hbm, cs_hbm, xq_hbm, xs_hbm, lab_hbm, ctab_hbm, xtab_hbm,
             out_hbm, idx_cq, idx_cs, idx_xq, idx_xs, lab_v,
             tiles_c, tiles_x, out_v, sems_c, sems_x):
    wid = lax.axis_index("s") * NC + lax.axis_index("c")
    base = wid * BPW

    ctab3 = ctab_hbm.reshape(VOCAB // SUB, SUB, DIM)
    xtab3 = xtab_hbm.reshape(VOCAB // SUB, SUB, DIM)

    pltpu.sync_copy(cq_hbm.at[pl.ds(base, BPW)], idx_cq)
    pltpu.sync_copy(cs_hbm.at[pl.ds(base, BPW)], idx_cs)
    pltpu.sync_copy(xq_hbm.at[pl.ds(base, BPW)], idx_xq)
    pltpu.sync_copy(xs_hbm.at[pl.ds(base, BPW)], idx_xs)
    pltpu.sync_copy(lab_hbm.at[pl.ds(base, BPW)], lab_v)

    lane = lax.iota(jnp.int32, L)

    def chunk_body(ch, acc):
        cbase = ch * CH

        def fire(g, carry):
            cq = idx_cq[pl.ds(cbase + g * L, L)]
            xq = idx_xq[pl.ds(cbase + g * L, L)]
            for r in range(L):
                i = g * L + r
                pltpu.make_async_copy(
                    ctab3.at[cq[r]], tiles_c.at[i], sems_c.at[r % NSEM]).start()
                pltpu.make_async_copy(
                    xtab3.at[xq[r]], tiles_x.at[i], sems_x.at[r % NSEM]).start()
            return carry

        lax.fori_loop(0, CH // L, fire, 0)
        for j in range(NSEM):
            pltpu.make_async_copy(
                ctab3.at[pl.ds(0, CH // NSEM)],
                tiles_c.at[pl.ds(0, CH // NSEM)], sems_c.at[j]).wait()
            pltpu.make_async_copy(
                xtab3.at[pl.ds(0, CH // NSEM)],
                tiles_x.at[pl.ds(0, CH // NSEM)], sems_x.at[j]).wait()

        def bce_body(g, acc):
            base_r = cbase + g * L
            cs = idx_cs[pl.ds(base_r, L)]
            xs = idx_xs[pl.ds(base_r, L)]
            s = jnp.zeros((L,), jnp.float32)
            for r in range(L):
                i = g * L + r
                sc_r = cs[r]
                sx_r = xs[r]
                prod = tiles_c[i, sc_r, pl.ds(0, L)] * tiles_x[i, sx_r, pl.ds(0, L)]
                for k in range(1, DIM // L):
                    prod = (prod + tiles_c[i, sc_r, pl.ds(k * L, L)]
                            * tiles_x[i, sx_r, pl.ds(k * L, L)])
                for sh in (8, 4, 2, 1):
                    prod = prod + prod.at[lane ^ sh].get(mode="promise_in_bounds")
                s = jnp.where(lane == r, prod, s)
            y = lab_v[pl.ds(base_r, L)]
            p = 1.0 / (1.0 + jnp.exp(-s))
            loss = -(y * _ln(p + 1e-8) + (1.0 - y) * _ln((1.0 - p) + 1e-8))
            return acc + loss

        return lax.fori_loop(0, CH // L, bce_body, acc)

    out_v[...] = lax.fori_loop(0, NCH, chunk_body, jnp.zeros((L,), jnp.float32))
    pltpu.sync_copy(out_v, out_hbm.at[pl.ds(wid * L, L)])


@jax.jit
def _run(center_ids, context_ids, labels, center_table, context_weights):
    mesh = plsc.VectorSubcoreMesh(core_axis_name="c", subcore_axis_name="s")
    cid = center_ids.astype(jnp.int32)
    xid = context_ids.astype(jnp.int32)
    partials = pl.kernel(
        _sc_body,
        out_type=jax.ShapeDtypeStruct((NW * L,), jnp.float32),
        mesh=mesh,
        compiler_params=pltpu.CompilerParams(
            needs_layout_passes=False, use_tc_tiling_on_sc=True),
        scratch_types=[
            pltpu.VMEM((BPW,), jnp.int32),
            pltpu.VMEM((BPW,), jnp.int32),
            pltpu.VMEM((BPW,), jnp.int32),
            pltpu.VMEM((BPW,), jnp.int32),
            pltpu.VMEM((BPW,), jnp.float32),
            pltpu.VMEM((CH, SUB, DIM), jnp.float32),
            pltpu.VMEM((CH, SUB, DIM), jnp.float32),
            pltpu.VMEM((L,), jnp.float32),
            pltpu.SemaphoreType.DMA((NSEM,)),
            pltpu.SemaphoreType.DMA((NSEM,)),
        ],
    )(
        cid >> 3,
        cid & 7,
        xid >> 3,
        xid & 7,
        labels,
        center_table,
        context_weights,
    )
    return jnp.sum(partials) / B


def kernel(center_ids, context_ids, labels, center_table, context_weights):
    return _run(center_ids, context_ids, labels, center_table, context_weights)


# speedup vs baseline: 1.0607x; 1.0607x over previous
import jax
import jax.numpy as jnp
from jax import lax
from jax.experimental import pallas as pl
from jax.experimental.pallas import tpu as pltpu
from jax.experimental.pallas import tpu_sc as plsc

VOCAB = 1000000
DIM = 64
B = 16384
SUB = 8

NC = 2
NS = 16
L = 16
NW = NC * NS
BPW = B // NW
CH = 32
NCH = BPW // CH

_LN2 = 0.6931471805599453
_SQRT2 = 1.4142135623730951


def _ln(x):
    bits = plsc.bitcast(x, jnp.int32)
    e = ((bits >> 23) & 0xFF) - 127
    m = plsc.bitcast((bits & 0x007FFFFF) | 0x3F800000, jnp.float32)
    big = m > _SQRT2
    m = jnp.where(big, m * 0.5, m)
    e = (e + jnp.where(big, 1, 0)).astype(jnp.float32)
    z = (m - 1.0) / (m + 1.0)
    z2 = z * z
    poly = 1.0 + z2 * (1.0 / 3.0 + z2 * (1.0 / 5.0 + z2 * (1.0 / 7.0 + z2 * (1.0 / 9.0))))
    return 2.0 * z * poly + e * _LN2


NSEM = 4


def _sc_body(cq_hbm, cs_hbm, xq_hbm, xs_hbm, lab_hbm, ctab_hbm, xtab_hbm,
             out_hbm, idx_cq, idx_cs, idx_xq, idx_xs, lab_v,
             rows_c, rows_x, out_v, sems_c, sems_x):
    wid = lax.axis_index("s") * NC + lax.axis_index("c")
    base = wid * BPW

    ctab3 = ctab_hbm.reshape(VOCAB // SUB, SUB, DIM)
    xtab3 = xtab_hbm.reshape(VOCAB // SUB, SUB, DIM)

    pltpu.sync_copy(cq_hbm.at[pl.ds(base, BPW)], idx_cq)
    pltpu.sync_copy(cs_hbm.at[pl.ds(base, BPW)], idx_cs)
    pltpu.sync_copy(xq_hbm.at[pl.ds(base, BPW)], idx_xq)
    pltpu.sync_copy(xs_hbm.at[pl.ds(base, BPW)], idx_xs)
    pltpu.sync_copy(lab_hbm.at[pl.ds(base, BPW)], lab_v)

    lane = lax.iota(jnp.int32, L)

    def chunk_body(ch, acc):
        cbase = ch * CH

        def fire(g, carry):
            cq = idx_cq[pl.ds(cbase + g * L, L)]
            xq = idx_xq[pl.ds(cbase + g * L, L)]
            cs = idx_cs[pl.ds(cbase + g * L, L)]
            xs = idx_xs[pl.ds(cbase + g * L, L)]
            for r in range(L):
                i = g * L + r
                pltpu.make_async_copy(
                    ctab3.at[cq[r], cs[r]], rows_c.at[i], sems_c.at[r % NSEM]).start()
                pltpu.make_async_copy(
                    xtab3.at[xq[r], xs[r]], rows_x.at[i], sems_x.at[r % NSEM]).start()
            return carry

        lax.fori_loop(0, CH // L, fire, 0)
        for j in range(NSEM):
            pltpu.make_async_copy(
                ctab3.at[0, pl.ds(0, CH // NSEM)],
                rows_c.at[pl.ds(0, CH // NSEM)], sems_c.at[j]).wait()
            pltpu.make_async_copy(
                xtab3.at[0, pl.ds(0, CH // NSEM)],
                rows_x.at[pl.ds(0, CH // NSEM)], sems_x.at[j]).wait()

        def bce_body(g, acc):
            base_r = cbase + g * L
            s = jnp.zeros((L,), jnp.float32)
            for r in range(L):
                i = g * L + r
                prod = rows_c[i, pl.ds(0, L)] * rows_x[i, pl.ds(0, L)]
                for k in range(1, DIM // L):
                    prod = (prod + rows_c[i, pl.ds(k * L, L)]
                            * rows_x[i, pl.ds(k * L, L)])
                for sh in (8, 4, 2, 1):
                    prod = prod + prod.at[lane ^ sh].get(mode="promise_in_bounds")
                s = jnp.where(lane == r, prod, s)
            y = lab_v[pl.ds(base_r, L)]
            p = 1.0 / (1.0 + jnp.exp(-s))
            loss = -(y * _ln(p + 1e-8) + (1.0 - y) * _ln((1.0 - p) + 1e-8))
            return acc + loss

        return lax.fori_loop(0, CH // L, bce_body, acc)

    out_v[...] = lax.fori_loop(0, NCH, chunk_body, jnp.zeros((L,), jnp.float32))
    pltpu.sync_copy(out_v, out_hbm.at[pl.ds(wid * L, L)])


@jax.jit
def _run(center_ids, context_ids, labels, center_table, context_weights):
    mesh = plsc.VectorSubcoreMesh(core_axis_name="c", subcore_axis_name="s")
    cid = center_ids.astype(jnp.int32)
    xid = context_ids.astype(jnp.int32)
    partials = pl.kernel(
        _sc_body,
        out_type=jax.ShapeDtypeStruct((NW * L,), jnp.float32),
        mesh=mesh,
        compiler_params=pltpu.CompilerParams(
            needs_layout_passes=False, use_tc_tiling_on_sc=True),
        scratch_types=[
            pltpu.VMEM((BPW,), jnp.int32),
            pltpu.VMEM((BPW,), jnp.int32),
            pltpu.VMEM((BPW,), jnp.int32),
            pltpu.VMEM((BPW,), jnp.int32),
            pltpu.VMEM((BPW,), jnp.float32),
            pltpu.VMEM((CH, DIM), jnp.float32),
            pltpu.VMEM((CH, DIM), jnp.float32),
            pltpu.VMEM((L,), jnp.float32),
            pltpu.SemaphoreType.DMA((NSEM,)),
            pltpu.SemaphoreType.DMA((NSEM,)),
        ],
    )(
        cid >> 3,
        cid & 7,
        xid >> 3,
        xid & 7,
        labels,
        center_table,
        context_weights,
    )
    return jnp.sum(partials) / B


def kernel(center_ids, context_ids, labels, center_table, context_weights):
    return _run(center_ids, context_ids, labels, center_table, context_weights)


# speedup vs baseline: 1.6119x; 1.5197x over previous
import jax
import jax.numpy as jnp
from jax import lax
from jax.experimental import pallas as pl
from jax.experimental.pallas import tpu as pltpu
from jax.experimental.pallas import tpu_sc as plsc

VOCAB = 1000000
DIM = 64
B = 16384
SUB = 8

NC = 2
NS = 16
L = 16
NW = NC * NS
BPW = B // NW
CH = 32
NCH = BPW // CH

_LN2 = 0.6931471805599453
_SQRT2 = 1.4142135623730951


def _ln(x):
    bits = plsc.bitcast(x, jnp.int32)
    e = ((bits >> 23) & 0xFF) - 127
    m = plsc.bitcast((bits & 0x007FFFFF) | 0x3F800000, jnp.float32)
    big = m > _SQRT2
    m = jnp.where(big, m * 0.5, m)
    e = (e + jnp.where(big, 1, 0)).astype(jnp.float32)
    z = (m - 1.0) / (m + 1.0)
    z2 = z * z
    poly = 1.0 + z2 * (1.0 / 3.0 + z2 * (1.0 / 5.0 + z2 * (1.0 / 7.0 + z2 * (1.0 / 9.0))))
    return 2.0 * z * poly + e * _LN2


NSEM = 4


def _sc_body(cq_hbm, cs_hbm, xq_hbm, xs_hbm, lab_hbm, ctab_hbm, xtab_hbm,
             out_hbm, idx_cq, idx_cs, idx_xq, idx_xs, lab_v,
             rows_c, rows_x, out_v, sems_c, sems_x):
    wid = lax.axis_index("s") * NC + lax.axis_index("c")
    base = wid * BPW

    ctab3 = ctab_hbm
    xtab3 = xtab_hbm

    pltpu.sync_copy(cq_hbm.at[pl.ds(base, BPW)], idx_cq)
    pltpu.sync_copy(cs_hbm.at[pl.ds(base, BPW)], idx_cs)
    pltpu.sync_copy(xq_hbm.at[pl.ds(base, BPW)], idx_xq)
    pltpu.sync_copy(xs_hbm.at[pl.ds(base, BPW)], idx_xs)
    pltpu.sync_copy(lab_hbm.at[pl.ds(base, BPW)], lab_v)

    lane = lax.iota(jnp.int32, L)

    def chunk_body(ch, acc):
        cbase = ch * CH

        def fire(g, carry):
            cq = idx_cq[pl.ds(cbase + g * L, L)]
            xq = idx_xq[pl.ds(cbase + g * L, L)]
            cs = idx_cs[pl.ds(cbase + g * L, L)]
            xs = idx_xs[pl.ds(cbase + g * L, L)]
            for r in range(L):
                i = g * L + r
                pltpu.make_async_copy(
                    ctab3.at[cq[r], cs[r]], rows_c.at[i], sems_c.at[r % NSEM]).start()
                pltpu.make_async_copy(
                    xtab3.at[xq[r], xs[r]], rows_x.at[i], sems_x.at[r % NSEM]).start()
            return carry

        lax.fori_loop(0, CH // L, fire, 0)
        for j in range(NSEM):
            pltpu.make_async_copy(
                ctab3.at[0, pl.ds(0, CH // NSEM)],
                rows_c.at[pl.ds(0, CH // NSEM)], sems_c.at[j]).wait()
            pltpu.make_async_copy(
                xtab3.at[0, pl.ds(0, CH // NSEM)],
                rows_x.at[pl.ds(0, CH // NSEM)], sems_x.at[j]).wait()

        def bce_body(g, acc):
            base_r = cbase + g * L
            s = jnp.zeros((L,), jnp.float32)
            for r in range(L):
                i = g * L + r
                prod = rows_c[i, pl.ds(0, L)] * rows_x[i, pl.ds(0, L)]
                for k in range(1, DIM // L):
                    prod = (prod + rows_c[i, pl.ds(k * L, L)]
                            * rows_x[i, pl.ds(k * L, L)])
                for sh in (8, 4, 2, 1):
                    prod = prod + prod.at[lane ^ sh].get(mode="promise_in_bounds")
                s = jnp.where(lane == r, prod, s)
            y = lab_v[pl.ds(base_r, L)]
            p = 1.0 / (1.0 + jnp.exp(-s))
            loss = -(y * _ln(p + 1e-8) + (1.0 - y) * _ln((1.0 - p) + 1e-8))
            return acc + loss

        return lax.fori_loop(0, CH // L, bce_body, acc)

    out_v[...] = lax.fori_loop(0, NCH, chunk_body, jnp.zeros((L,), jnp.float32))
    pltpu.sync_copy(out_v, out_hbm.at[pl.ds(wid * L, L)])


@jax.jit
def _run(center_ids, context_ids, labels, center_table, context_weights):
    mesh = plsc.VectorSubcoreMesh(core_axis_name="c", subcore_axis_name="s")
    cid = center_ids.astype(jnp.int32)
    xid = context_ids.astype(jnp.int32)
    partials = pl.kernel(
        _sc_body,
        out_type=jax.ShapeDtypeStruct((NW * L,), jnp.float32),
        mesh=mesh,
        compiler_params=pltpu.CompilerParams(
            needs_layout_passes=False, use_tc_tiling_on_sc=True),
        scratch_types=[
            pltpu.VMEM((BPW,), jnp.int32),
            pltpu.VMEM((BPW,), jnp.int32),
            pltpu.VMEM((BPW,), jnp.int32),
            pltpu.VMEM((BPW,), jnp.int32),
            pltpu.VMEM((BPW,), jnp.float32),
            pltpu.VMEM((CH, DIM), jnp.float32),
            pltpu.VMEM((CH, DIM), jnp.float32),
            pltpu.VMEM((L,), jnp.float32),
            pltpu.SemaphoreType.DMA((NSEM,)),
            pltpu.SemaphoreType.DMA((NSEM,)),
        ],
    )(
        cid >> 3,
        cid & 7,
        xid >> 3,
        xid & 7,
        labels,
        center_table.reshape(VOCAB // SUB, SUB, DIM),
        context_weights.reshape(VOCAB // SUB, SUB, DIM),
    )
    return jnp.sum(partials) / B


def kernel(center_ids, context_ids, labels, center_table, context_weights):
    return _run(center_ids, context_ids, labels, center_table, context_weights)
